# baseline (speedup 1.0000x reference)
import jax
import jax.numpy as jnp
from jax.experimental import pallas as pl
from jax.experimental.pallas import tpu as pltpu

M, D = 131072, 128
TO_ADD_CONST = 1.0
CHUNK = 4096
NCHUNKS = M // CHUNK
NBUF = 8


def _body(idx_ref, x_hbm, o_hbm, *rest):
    bufs = rest[:NBUF]
    in_sems = rest[NBUF:2 * NBUF]
    out_sems = rest[2 * NBUF:3 * NBUF]
    idx = idx_ref[0]

    def in_cp(k):
        b = k % NBUF
        return pltpu.make_async_copy(
            x_hbm.at[pl.ds(k * CHUNK, CHUNK), :], bufs[b], in_sems[b])

    def out_cp(k):
        b = k % NBUF
        return pltpu.make_async_copy(
            bufs[b], o_hbm.at[pl.ds(k * CHUNK, CHUNK), :], out_sems[b])

    for j in range(NBUF):
        in_cp(j).start()

    for k in range(NCHUNKS):
        in_cp(k).wait()
        b = k % NBUF
        base = k * CHUNK

        @pl.when((idx >= base) & (idx < base + CHUNK))
        def _(b=b, base=base):
            r = idx - base
            bufs[b][pl.ds(r, 1), :] = bufs[b][pl.ds(r, 1), :] + TO_ADD_CONST

        out_cp(k).start()
        j = k + NBUF
        if j < NCHUNKS:
            out_cp(k).wait()
            in_cp(j).start()

    for k in range(NCHUNKS - NBUF, NCHUNKS):
        out_cp(k).wait()


@jax.jit
def _run(tensor, idx_arr):
    grid_spec = pltpu.PrefetchScalarGridSpec(
        num_scalar_prefetch=1,
        grid=(1,),
        in_specs=[pl.BlockSpec(memory_space=pl.ANY)],
        out_specs=pl.BlockSpec(memory_space=pl.ANY),
        scratch_shapes=(
            [pltpu.VMEM((CHUNK, D), jnp.float32)] * NBUF
            + [pltpu.SemaphoreType.DMA] * (2 * NBUF)
        ),
    )
    return pl.pallas_call(
        _body,
        grid_spec=grid_spec,
        out_shape=jax.ShapeDtypeStruct((M, D), jnp.float32),
    )(idx_arr, tensor)


def kernel(tensor, slice_index, related_index):
    idx_arr = jnp.asarray(slice_index, dtype=jnp.int32).reshape((1,))
    out = _run(tensor, idx_arr)
    return (out, slice_index, related_index)


# speedup vs baseline: 1.0430x; 1.0430x over previous
import jax
import jax.numpy as jnp
from jax.experimental import pallas as pl
from jax.experimental.pallas import tpu as pltpu

M, D = 131072, 128
TO_ADD_CONST = 1.0
CHUNK = 8192
NCHUNKS = M // CHUNK
NBUF = 4


def _body(idx_ref, x_hbm, o_hbm, *rest):
    bufs = rest[:NBUF]
    in_sems = rest[NBUF:2 * NBUF]
    out_sems = rest[2 * NBUF:3 * NBUF]
    idx = idx_ref[0]

    def in_cp(k):
        b = k % NBUF
        return pltpu.make_async_copy(
            x_hbm.at[pl.ds(k * CHUNK, CHUNK), :], bufs[b], in_sems[b])

    def out_cp(k):
        b = k % NBUF
        return pltpu.make_async_copy(
            bufs[b], o_hbm.at[pl.ds(k * CHUNK, CHUNK), :], out_sems[b])

    for j in range(NBUF):
        in_cp(j).start()

    for k in range(NCHUNKS):
        in_cp(k).wait()
        b = k % NBUF
        base = k * CHUNK

        @pl.when((idx >= base) & (idx < base + CHUNK))
        def _(b=b, base=base):
            r = idx - base
            bufs[b][pl.ds(r, 1), :] = bufs[b][pl.ds(r, 1), :] + TO_ADD_CONST

        out_cp(k).start()
        j = k + NBUF
        if j < NCHUNKS:
            out_cp(k).wait()
            in_cp(j).start()

    for k in range(NCHUNKS - NBUF, NCHUNKS):
        out_cp(k).wait()


@jax.jit
def _run(tensor, idx_arr):
    grid_spec = pltpu.PrefetchScalarGridSpec(
        num_scalar_prefetch=1,
        grid=(1,),
        in_specs=[pl.BlockSpec(memory_space=pl.ANY)],
        out_specs=pl.BlockSpec(memory_space=pl.ANY),
        scratch_shapes=(
            [pltpu.VMEM((CHUNK, D), jnp.float32)] * NBUF
            + [pltpu.SemaphoreType.DMA] * (2 * NBUF)
        ),
    )
    return pl.pallas_call(
        _body,
        grid_spec=grid_spec,
        out_shape=jax.ShapeDtypeStruct((M, D), jnp.float32),
    )(idx_arr, tensor)


def kernel(tensor, slice_index, related_index):
    idx_arr = jnp.asarray(slice_index, dtype=jnp.int32).reshape((1,))
    out = _run(tensor, idx_arr)
    return (out, slice_index, related_index)


# speedup vs baseline: 1.0615x; 1.0177x over previous
import jax
import jax.numpy as jnp
from jax.experimental import pallas as pl
from jax.experimental.pallas import tpu as pltpu

M, D = 131072, 128
TO_ADD_CONST = 1.0
CHUNK = 8192
NCHUNKS = M // CHUNK
NBUF = 6


def _body(idx_ref, x_hbm, o_hbm, *rest):
    bufs = rest[:NBUF]
    in_sems = rest[NBUF:2 * NBUF]
    out_sems = rest[2 * NBUF:3 * NBUF]
    idx = idx_ref[0]

    def in_cp(k):
        b = k % NBUF
        return pltpu.make_async_copy(
            x_hbm.at[pl.ds(k * CHUNK, CHUNK), :], bufs[b], in_sems[b])

    def out_cp(k):
        b = k % NBUF
        return pltpu.make_async_copy(
            bufs[b], o_hbm.at[pl.ds(k * CHUNK, CHUNK), :], out_sems[b])

    for j in range(NBUF):
        in_cp(j).start()

    for k in range(NCHUNKS):
        in_cp(k).wait()
        b = k % NBUF
        base = k * CHUNK

        @pl.when((idx >= base) & (idx < base + CHUNK))
        def _(b=b, base=base):
            r = idx - base
            bufs[b][pl.ds(r, 1), :] = bufs[b][pl.ds(r, 1), :] + TO_ADD_CONST

        out_cp(k).start()
        j = k + NBUF
        if j < NCHUNKS:
            out_cp(k).wait()
            in_cp(j).start()

    for k in range(NCHUNKS - NBUF, NCHUNKS):
        out_cp(k).wait()


@jax.jit
def _run(tensor, idx_arr):
    grid_spec = pltpu.PrefetchScalarGridSpec(
        num_scalar_prefetch=1,
        grid=(1,),
        in_specs=[pl.BlockSpec(memory_space=pl.ANY)],
        out_specs=pl.BlockSpec(memory_space=pl.ANY),
        scratch_shapes=(
            [pltpu.VMEM((CHUNK, D), jnp.float32)] * NBUF
            + [pltpu.SemaphoreType.DMA] * (2 * NBUF)
        ),
    )
    return pl.pallas_call(
        _body,
        grid_spec=grid_spec,
        out_shape=jax.ShapeDtypeStruct((M, D), jnp.float32),
    )(idx_arr, tensor)


def kernel(tensor, slice_index, related_index):
    idx_arr = jnp.asarray(slice_index, dtype=jnp.int32).reshape((1,))
    out = _run(tensor, idx_arr)
    return (out, slice_index, related_index)
